# baseline (speedup 1.0000x reference)
import functools

import jax
import jax.numpy as jnp
from jax import lax
from jax.experimental import pallas as pl
from jax.experimental.pallas import tpu as pltpu
from jax.experimental.pallas import tpu_sc as plsc

MAXLEN = 200
EMBED = 64
PADE = 128
BATCH = 1024
VOCAB = 100000

NUM_CORES = 2
NUM_SUBCORES = 16
NUM_WORKERS = NUM_CORES * NUM_SUBCORES
SEQ_PER_W = BATCH // NUM_WORKERS
LANES = 16

_UNPACK_COLS = [16 * j for j in range(MAXLEN // 16)] + [MAXLEN - 16]


def _make_kernel():
    mesh = plsc.VectorSubcoreMesh(core_axis_name="c", subcore_axis_name="s")

    @functools.partial(
        pl.kernel,
        mesh=mesh,
        out_type=jax.ShapeDtypeStruct((BATCH, MAXLEN, EMBED), jnp.float32),
        scratch_types=[
            pltpu.VMEM((8, MAXLEN), jnp.int32),
            [pltpu.VMEM((MAXLEN,), jnp.int32)] * SEQ_PER_W,
            pltpu.VMEM((MAXLEN * EMBED,), jnp.float32),
            [pltpu.VMEM((MAXLEN, PADE), jnp.float32)] * 2,
            [pltpu.VMEM((MAXLEN, EMBED), jnp.float32)] * 2,
            [pltpu.SemaphoreType.DMA] * 2,
            [pltpu.SemaphoreType.DMA] * 2,
        ],
        compiler_params=pltpu.CompilerParams(use_tc_tiling_on_sc=True),
    )
    def emb_kernel(x_hbm, tok_hbm, pos_hbm, out_hbm, stage_v, idx_v, pos_v,
                   gbufs, obufs, gsems, ssems):
        wid = lax.axis_index("s") * NUM_CORES + lax.axis_index("c")
        seq0 = wid * SEQ_PER_W
        pltpu.sync_copy(pos_hbm, pos_v)
        for r in range(SEQ_PER_W // 8):
            pltpu.sync_copy(x_hbm.at[pl.ds(seq0 + 8 * r, 8)], stage_v)
            for s8 in range(8):
                for c in _UNPACK_COLS:
                    idx_v[8 * r + s8][pl.ds(c, LANES)] = (
                        stage_v[s8, pl.ds(c, LANES)])

        def fire_gather(s):
            pltpu.async_copy(tok_hbm.at[idx_v[s]], gbufs[s % 2], gsems[s % 2])

        def add_seq(s):
            gb, ob = gbufs[s % 2], obufs[s % 2]

            def body(p, c):
                for j in range(EMBED // LANES):
                    sl = pl.ds(j * LANES, LANES)
                    ob[p, sl] = (gb[p, sl]
                                 + pos_v[pl.ds(p * EMBED + j * LANES, LANES)])
                return c

            lax.fori_loop(0, MAXLEN, body, 0)

        fire_gather(0)
        for s in range(SEQ_PER_W):
            if s + 1 < SEQ_PER_W:
                fire_gather(s + 1)
            pltpu.make_async_copy(
                tok_hbm.at[idx_v[s]], gbufs[s % 2], gsems[s % 2]).wait()
            if s >= 2:
                pltpu.make_async_copy(
                    obufs[s % 2], out_hbm.at[seq0 + s - 2], ssems[s % 2]
                ).wait()
            add_seq(s)
            pltpu.async_copy(obufs[s % 2], out_hbm.at[seq0 + s], ssems[s % 2])
        for s in (SEQ_PER_W - 2, SEQ_PER_W - 1):
            pltpu.make_async_copy(
                obufs[s % 2], out_hbm.at[seq0 + s], ssems[s % 2]).wait()

    return emb_kernel


_emb = _make_kernel()


def kernel(x, token_table, pos_table):
    tok128 = jnp.pad(token_table, ((0, 0), (0, PADE - EMBED)))
    return _emb(x.astype(jnp.int32), tok128, pos_table.reshape(-1))


# speedup vs baseline: 1.5958x; 1.5958x over previous
import functools

import jax
import jax.numpy as jnp
from jax import lax
from jax.experimental import pallas as pl
from jax.experimental.pallas import tpu as pltpu
from jax.experimental.pallas import tpu_sc as plsc

MAXLEN = 200
EMBED = 64
BATCH = 1024
OUTMIN = 128

NUM_CORES = 2
NUM_SUBCORES = 16
NUM_WORKERS = NUM_CORES * NUM_SUBCORES
SEQ_PER_W = BATCH // NUM_WORKERS
LANES = 16

GRP = 2
NGRP = SEQ_PER_W // GRP
NBUF = 4
LEAD = 2
LAG = 2


def _make_kernel():
    mesh = plsc.VectorSubcoreMesh(core_axis_name="c", subcore_axis_name="s")

    @functools.partial(
        pl.kernel,
        mesh=mesh,
        out_type=jax.ShapeDtypeStruct((BATCH, MAXLEN, OUTMIN), jnp.float32),
        scratch_types=[
            [pltpu.VMEM((MAXLEN,), jnp.int32)] * SEQ_PER_W,
            pltpu.VMEM((MAXLEN, EMBED), jnp.float32),
            [pltpu.VMEM((GRP, MAXLEN, EMBED), jnp.float32)] * NBUF,
            [pltpu.SemaphoreType.DMA] * NBUF,
            [pltpu.SemaphoreType.DMA] * NBUF,
        ],
        compiler_params=pltpu.CompilerParams(use_tc_tiling_on_sc=False),
    )
    def emb_kernel(x_hbm, tok_hbm, pos_hbm, out_hbm, idx_v, pos_v, bufs,
                   gsems, ssems):
        wid = lax.axis_index("s") * NUM_CORES + lax.axis_index("c")
        seq0 = wid * SEQ_PER_W
        for s in range(SEQ_PER_W):
            pltpu.async_copy(
                x_hbm.at[pl.ds((seq0 + s) * MAXLEN, MAXLEN)], idx_v[s],
                gsems[0])
        for s in range(SEQ_PER_W):
            pltpu.make_async_copy(
                x_hbm.at[pl.ds((seq0 + s) * MAXLEN, MAXLEN)], idx_v[s],
                gsems[0]).wait()
        pltpu.sync_copy(pos_hbm, pos_v)

        def fire_gathers(t):
            b = t % NBUF
            for k in range(GRP):
                pltpu.async_copy(
                    tok_hbm.at[idx_v[t * GRP + k]], bufs[b].at[k], gsems[b])

        def add_group(t):
            b = t % NBUF
            buf = bufs[b]

            def body(p, c):
                pos_regs = [pos_v[p, pl.ds(j * LANES, LANES)]
                            for j in range(EMBED // LANES)]
                for k in range(GRP):
                    for j in range(EMBED // LANES):
                        sl = pl.ds(j * LANES, LANES)
                        buf[k, p, sl] = buf[k, p, sl] + pos_regs[j]
                return c

            lax.fori_loop(0, MAXLEN, body, 0)

        def out_slice(t):
            return out_hbm.at[pl.ds(seq0 + t * GRP, GRP), :, pl.ds(0, EMBED)]

        for t in range(LEAD):
            fire_gathers(t)
        for t in range(NGRP):
            b = t % NBUF
            for k in range(GRP):
                pltpu.make_async_copy(
                    tok_hbm.at[idx_v[t * GRP + k]], bufs[b].at[k],
                    gsems[b]).wait()
            add_group(t)
            pltpu.async_copy(bufs[b], out_slice(t), ssems[b])
            if t >= LAG:
                ob = (t - LAG) % NBUF
                pltpu.make_async_copy(
                    bufs[ob], out_slice(t - LAG), ssems[ob]).wait()
            if t + LEAD < NGRP:
                fire_gathers(t + LEAD)
        for t in range(NGRP - LAG, NGRP):
            b = t % NBUF
            pltpu.make_async_copy(bufs[b], out_slice(t), ssems[b]).wait()

    return emb_kernel


_emb = _make_kernel()

_UNPACK_COLS = [16 * j for j in range(MAXLEN // 16)] + [MAXLEN - 16]


def _make_flatten():
    mesh = plsc.VectorSubcoreMesh(core_axis_name="c", subcore_axis_name="s")

    @functools.partial(
        pl.kernel,
        mesh=mesh,
        out_type=jax.ShapeDtypeStruct((BATCH * MAXLEN,), jnp.int32),
        scratch_types=[
            pltpu.VMEM((SEQ_PER_W, MAXLEN), jnp.int32),
            pltpu.VMEM((SEQ_PER_W * MAXLEN,), jnp.int32),
        ],
        compiler_params=pltpu.CompilerParams(use_tc_tiling_on_sc=True),
    )
    def flatten_kernel(x_hbm, out_hbm, stage_v, flat_v):
        wid = lax.axis_index("s") * NUM_CORES + lax.axis_index("c")
        seq0 = wid * SEQ_PER_W
        pltpu.sync_copy(x_hbm.at[pl.ds(seq0, SEQ_PER_W)], stage_v)
        for s in range(SEQ_PER_W):
            for c in _UNPACK_COLS:
                flat_v[pl.ds(s * MAXLEN + c, LANES)] = (
                    stage_v[s, pl.ds(c, LANES)])
        pltpu.sync_copy(
            flat_v, out_hbm.at[pl.ds(seq0 * MAXLEN, SEQ_PER_W * MAXLEN)])

    return flatten_kernel


_flatten_x = _make_flatten()


def kernel(x, token_table, pos_table):
    xflat = _flatten_x(x.astype(jnp.int32))
    out = _emb(xflat, token_table, pos_table)
    return out[:, :, :EMBED]
